# baseline (speedup 1.0000x reference)
import functools
import math

import jax
import jax.numpy as jnp
from jax import lax
from jax.experimental import pallas as pl
from jax.experimental.pallas import tpu as pltpu
from jax.experimental.pallas import tpu_sc as plsc

D = 64
BATCH = 4096
SEQ = 200
B = BATCH * SEQ
NC = 2
NS = 16
NW = NC * NS
BLK = 128
NBT = BATCH // BLK
NBLK = B // BLK
BLK_PER_W = NBLK // NW
NBUF = 4
LOOKAHEAD = 2
BSTRIDE = 72
SCALE = math.sqrt(D)


def _body(idx_hbm, table_hbm, out_hbm, idx_v,
          b0, b1, b2, b3, t0, t1, g0, g1, g2, g3, w0, w1):
    bufs = [b0, b1, b2, b3]
    tbufs = [t0, t1]
    gsems = [g0, g1, g2, g3]
    wsems = [w0, w1]
    wid = lax.axis_index("s") * NC + lax.axis_index("c")
    blk0 = wid * BLK_PER_W
    pltpu.sync_copy(idx_hbm.at[pl.ds(blk0 * BLK, BLK_PER_W * BLK)], idx_v)

    def start_gather(g, bi):
        pltpu.async_copy(
            table_hbm.at[idx_v.at[pl.ds(g * BLK, BLK)]], bufs[bi], gsems[bi]
        )

    def wait_gather(bi):
        pltpu.make_async_copy(
            table_hbm.at[idx_v.at[pl.ds(0, BLK)]], bufs[bi], gsems[bi]
        ).wait()

    def start_write(g, ti):
        n = blk0 + g
        st = n // 256
        rem = n - st * 256
        bt = rem // 8
        si = rem - bt * 8
        s = st * 8 + si
        pltpu.async_copy(tbufs[ti], out_hbm.at[s, :, bt], wsems[ti])

    def wait_write(ti):
        pltpu.make_async_copy(
            tbufs[ti], out_hbm.at[0, :, 0], wsems[ti]
        ).wait()

    iota = lax.iota(jnp.int32, 16)

    def transpose_scale(bi, ti):
        buf = bufs[bi]
        tbuf = tbufs[ti]
        for j in range(8):
            b_idx = iota + (j * 16)

            @plsc.parallel_loop(0, D, 1, unroll=16)
            def _col_block(cc):
                ct = cc // 8
                cw = cc - ct * 8
                v = plsc.load_gather(buf, [b_idx, jnp.full((16,), cc, jnp.int32)])
                tbuf[ct, cw, pl.ds(j * 16, 16)] = v * SCALE

    for g in range(LOOKAHEAD):
        start_gather(g, g)

    def group_body(gg, _):
        for bi in range(NBUF):
            g = gg * NBUF + bi
            ti = bi % 2
            wait_gather(bi)

            @pl.when(g >= 2)
            def _():
                wait_write(ti)

            transpose_scale(bi, ti)
            start_write(g, ti)
            gn = g + LOOKAHEAD
            bn = (bi + LOOKAHEAD) % NBUF

            @pl.when(gn < BLK_PER_W)
            def _():
                start_gather(gn, bn)

        return 0

    lax.fori_loop(0, BLK_PER_W // NBUF, group_body, 0)

    for ti in range(2):
        wait_write(ti)


_gather_scale = functools.partial(
    pl.kernel,
    mesh=plsc.VectorSubcoreMesh(core_axis_name="c", subcore_axis_name="s"),
    out_type=jax.ShapeDtypeStruct((SEQ, D // 8, NBT, 8, BLK), jnp.float32),
    scratch_types=(
        [pltpu.VMEM((BLK_PER_W * BLK,), jnp.int32)]
        + [pltpu.VMEM((BLK, D), jnp.float32) for _ in range(NBUF)]
        + [pltpu.VMEM((D // 8, 8, BLK), jnp.float32) for _ in range(2)]
        + [pltpu.SemaphoreType.DMA for _ in range(NBUF + 2)]
    ),
    compiler_params=pltpu.CompilerParams(
        use_tc_tiling_on_sc=False, needs_layout_passes=False
    ),
)(_body)


def kernel(node_ids, order_ids, value_ids, embedding_weight):
    idx = (
        node_ids.T.reshape(SEQ // 8, 8, BATCH // BLK, BLK)
        .transpose(0, 2, 1, 3)
        .reshape(-1)
    )
    out5 = _gather_scale(idx, embedding_weight)
    return jnp.transpose(out5, (2, 4, 0, 1, 3)).reshape(BATCH, SEQ, D)


# speedup vs baseline: 1.0373x; 1.0373x over previous
import functools
import math

import jax
import jax.numpy as jnp
from jax import lax
from jax.experimental import pallas as pl
from jax.experimental.pallas import tpu as pltpu
from jax.experimental.pallas import tpu_sc as plsc

D = 64
BATCH = 4096
SEQ = 200
B = BATCH * SEQ
NC = 2
NS = 16
NW = NC * NS
BLK = 128
NBT = BATCH // BLK
NBLK = B // BLK
BLK_PER_W = NBLK // NW
NBUF = 4
LOOKAHEAD = 2
SCALE = math.sqrt(D)


def _body(idx_hbm, table_hbm, out_hbm, idx_v,
          b0, b1, b2, b3, t0, t1, g0, g1, g2, g3, w0, w1):
    bufs = [b0, b1, b2, b3]
    tbufs = [t0, t1]
    gsems = [g0, g1, g2, g3]
    wsems = [w0, w1]
    wid = lax.axis_index("s") * NC + lax.axis_index("c")
    blk0 = wid * BLK_PER_W
    pltpu.sync_copy(idx_hbm.at[pl.ds(blk0 * BLK, BLK_PER_W * BLK)], idx_v)

    def start_gather(g, bi):
        pltpu.async_copy(
            table_hbm.at[idx_v.at[pl.ds(g * BLK, BLK)]], bufs[bi], gsems[bi]
        )

    def wait_gather(bi):
        pltpu.make_async_copy(
            table_hbm.at[idx_v.at[pl.ds(0, BLK)]], bufs[bi], gsems[bi]
        ).wait()

    def start_write(g, ti):
        n = blk0 + g
        st = n // 256
        rem = n - st * 256
        bt = rem // 8
        si = rem - bt * 8
        s = st * 8 + si
        pltpu.async_copy(tbufs[ti], out_hbm.at[s, :, bt], wsems[ti])

    def wait_write(ti):
        pltpu.make_async_copy(
            tbufs[ti], out_hbm.at[0, :, 0], wsems[ti]
        ).wait()

    iota = lax.iota(jnp.int32, 16)

    def transpose_scale(bi, ti):
        buf = bufs[bi]
        tbuf = tbufs[ti]

        @plsc.parallel_loop(0, D, 1, unroll=4)
        def _col_block(cc):
            ct = cc // 8
            cw = cc - ct * 8
            c_idx = jnp.full((16,), cc, jnp.int32)
            for j in range(8):
                b_idx = iota + (j * 16)
                v = plsc.load_gather(buf, [b_idx, c_idx])
                tbuf[ct, cw, pl.ds(j * 16, 16)] = v * SCALE

    for g in range(LOOKAHEAD):
        start_gather(g, g)

    def group_body(gg, _):
        for bi in range(NBUF):
            g = gg * NBUF + bi
            ti = bi % 2
            wait_gather(bi)

            @pl.when(g >= 2)
            def _():
                wait_write(ti)

            transpose_scale(bi, ti)
            start_write(g, ti)
            gn = g + LOOKAHEAD
            bn = (bi + LOOKAHEAD) % NBUF

            @pl.when(gn < BLK_PER_W)
            def _():
                start_gather(gn, bn)

        return 0

    lax.fori_loop(0, BLK_PER_W // NBUF, group_body, 0)

    for ti in range(2):
        wait_write(ti)


_gather_scale = functools.partial(
    pl.kernel,
    mesh=plsc.VectorSubcoreMesh(core_axis_name="c", subcore_axis_name="s"),
    out_type=jax.ShapeDtypeStruct((SEQ, D // 8, NBT, 8, BLK), jnp.float32),
    scratch_types=(
        [pltpu.VMEM((BLK_PER_W * BLK,), jnp.int32)]
        + [pltpu.VMEM((BLK, D), jnp.float32) for _ in range(NBUF)]
        + [pltpu.VMEM((D // 8, 8, BLK), jnp.float32) for _ in range(2)]
        + [pltpu.SemaphoreType.DMA for _ in range(NBUF + 2)]
    ),
    compiler_params=pltpu.CompilerParams(
        use_tc_tiling_on_sc=False, needs_layout_passes=False
    ),
)(_body)


def kernel(node_ids, order_ids, value_ids, embedding_weight):
    idx = (
        node_ids.T.reshape(SEQ // 8, 8, BATCH // BLK, BLK)
        .transpose(0, 2, 1, 3)
        .reshape(-1)
    )
    out5 = _gather_scale(idx, embedding_weight)
    return jnp.transpose(out5, (2, 4, 0, 1, 3)).reshape(BATCH, SEQ, D)
